# baseline (speedup 1.0000x reference)
import functools

import jax
import jax.numpy as jnp
from jax import lax
from jax.experimental import pallas as pl
from jax.experimental.pallas import tpu as pltpu
from jax.experimental.pallas import tpu_sc as plsc

N = 10000
E = 640000
D = 64
DT = 64
H = 128
DE = 15
NC = 2
NS = 16
NW = NC * NS
EPW = E // NW
CH = 80
NCH = EPW // CH
NP = 10240
RPT = NP // NS
BE = 1024
GE = E // BE
BN = 1000
GN = N // BN


def _sc_gather_body(tab_hbm, src_hbm, dst_hbm, hps_out, hpd_out,
                    idx_s, idx_d, bs0, bd0, bs1, bd1, sem0, sem1):
    c = lax.axis_index("c")
    s = lax.axis_index("s")
    wid = s * NC + c
    half = wid // (NW // 2)
    widh = wid % (NW // 2)
    lane0 = half * DT
    pltpu.sync_copy(src_hbm.at[wid], idx_s)
    pltpu.sync_copy(dst_hbm.at[wid], idx_d)
    bufs = ((bs0, bd0, sem0), (bs1, bd1, sem1))

    def start(g, b):
        bs, bd, sem = bufs[b]
        pltpu.make_async_copy(tab_hbm.at[idx_s.at[g]], bs, sem).start()
        pltpu.make_async_copy(tab_hbm.at[idx_d.at[g]], bd, sem).start()

    def finish(g, b):
        bs, bd, sem = bufs[b]
        pltpu.make_async_copy(tab_hbm.at[idx_s.at[g]], bs, sem).wait()
        pltpu.make_async_copy(tab_hbm.at[idx_d.at[g]], bd, sem).wait()
        r0 = widh * EPW + g * CH
        pltpu.sync_copy(bs, hps_out.at[pl.ds(r0, CH), pl.ds(lane0, DT)])
        pltpu.sync_copy(bd, hpd_out.at[pl.ds(r0, CH), pl.ds(lane0, DT)])

    start(0, 0)

    def body(i, carry):
        g0 = 2 * i
        start(g0 + 1, 1)
        finish(g0, 0)

        @pl.when(i < NCH // 2 - 1)
        def _():
            start(g0 + 2, 0)

        finish(g0 + 1, 1)
        return carry

    lax.fori_loop(0, NCH // 2, body, 0)


def _sc_scatter_body(with_w, *refs):
    if with_w:
        (m2e_hbm, m2o_hbm, wde_hbm, wdo_hbm, dste_hbm, dsto_hbm,
         zm_hbm, za_hbm, outm, outa,
         idx0, idx1, m0, m1, w0, w1, macc, aacc, sem0, sem1) = refs
        bufs = ((idx0, m0, w0, sem0), (idx1, m1, w1, sem1))
    else:
        (m2e_hbm, m2o_hbm, dste_hbm, dsto_hbm, zm_hbm, outm,
         idx0, idx1, m0, m1, macc, sem0, sem1) = refs
        wde_hbm = wdo_hbm = None
        bufs = ((idx0, m0, None, sem0), (idx1, m1, None, sem1))
    c = lax.axis_index("c")
    s = lax.axis_index("s")
    wid = s * NC + c
    pltpu.sync_copy(zm_hbm, macc.at[pl.ds(s * RPT, RPT)])
    if with_w:
        pltpu.sync_copy(za_hbm, aacc.at[pl.ds(s * RPT, RPT)])
    plsc.subcore_barrier()
    epw_h = EPW // 2

    def run_pass(m2_hbm, wd_hbm, dst_hbm):
        def start(g, b):
            iv, mv, wv, sem = bufs[b]
            e0 = wid * epw_h + g * CH
            pltpu.make_async_copy(dst_hbm.at[wid].at[pl.ds(g, 1)],
                                  iv, sem).start()
            pltpu.make_async_copy(m2_hbm.at[pl.ds(e0, CH)], mv, sem).start()
            if with_w:
                pltpu.make_async_copy(wd_hbm.at[pl.ds(e0, CH)],
                                      wv, sem).start()

        def finish(g, b):
            iv, mv, wv, sem = bufs[b]
            e0 = wid * epw_h + g * CH
            pltpu.make_async_copy(dst_hbm.at[wid].at[pl.ds(g, 1)],
                                  iv, sem).wait()
            pltpu.make_async_copy(m2_hbm.at[pl.ds(e0, CH)], mv, sem).wait()
            if with_w:
                pltpu.make_async_copy(wd_hbm.at[pl.ds(e0, CH)],
                                      wv, sem).wait()
            pltpu.sync_copy(mv, macc.at[iv.at[0]], add=True)
            if with_w:
                pltpu.sync_copy(wv, aacc.at[iv.at[0]], add=True)

        nch_h = NCH // 2
        start(0, 0)

        def body(i, carry):
            g0 = 2 * i

            @pl.when(g0 + 1 < nch_h)
            def _():
                start(g0 + 1, 1)

            finish(g0, 0)

            @pl.when(g0 + 2 < nch_h)
            def _():
                start(g0 + 2, 0)

            @pl.when(g0 + 1 < nch_h)
            def _():
                finish(g0 + 1, 1)

            return carry

        lax.fori_loop(0, (nch_h + 1) // 2, body, 0)

    run_pass(m2e_hbm, wde_hbm, dste_hbm)
    run_pass(m2o_hbm, wdo_hbm, dsto_hbm)
    plsc.subcore_barrier()
    pltpu.sync_copy(macc.at[pl.ds(s * RPT, RPT)],
                    outm.at[c].at[pl.ds(s * RPT, RPT)])
    if with_w:
        pltpu.sync_copy(aacc.at[pl.ds(s * RPT, RPT)],
                        outa.at[c].at[pl.ds(s * RPT, RPT)])


@functools.lru_cache(maxsize=None)
def _sc_mesh():
    return plsc.VectorSubcoreMesh(core_axis_name="c", subcore_axis_name="s")


@functools.lru_cache(maxsize=None)
def _sc_gather():
    return pl.kernel(
        _sc_gather_body,
        mesh=_sc_mesh(),
        out_type=[
            jax.ShapeDtypeStruct((E // 2, 2 * DT), jnp.float32),
            jax.ShapeDtypeStruct((E // 2, 2 * DT), jnp.float32),
        ],
        scratch_types=[
            pltpu.VMEM((NCH, CH), jnp.int32),
            pltpu.VMEM((NCH, CH), jnp.int32),
            pltpu.VMEM((CH, DT), jnp.float32),
            pltpu.VMEM((CH, DT), jnp.float32),
            pltpu.VMEM((CH, DT), jnp.float32),
            pltpu.VMEM((CH, DT), jnp.float32),
            pltpu.SemaphoreType.DMA,
            pltpu.SemaphoreType.DMA,
        ],
        compiler_params=pltpu.CompilerParams(use_tc_tiling_on_sc=False),
    )


@functools.lru_cache(maxsize=None)
def _sc_scatter(with_w):
    out_type = [jax.ShapeDtypeStruct((NC, NP, H), jnp.float32)]
    scratch = [
        pltpu.VMEM((1, CH), jnp.int32),
        pltpu.VMEM((1, CH), jnp.int32),
        pltpu.VMEM((CH, H), jnp.float32),
        pltpu.VMEM((CH, H), jnp.float32),
    ]
    if with_w:
        out_type.append(jax.ShapeDtypeStruct((NC, NP, 16), jnp.float32))
        scratch.append(pltpu.VMEM((CH, 16), jnp.float32))
        scratch.append(pltpu.VMEM((CH, 16), jnp.float32))
    scratch.append(pltpu.VMEM_SHARED((NP, H), jnp.float32))
    if with_w:
        scratch.append(pltpu.VMEM_SHARED((NP, 16), jnp.float32))
    scratch.append(pltpu.SemaphoreType.DMA)
    scratch.append(pltpu.SemaphoreType.DMA)
    return pl.kernel(
        functools.partial(_sc_scatter_body, with_w),
        mesh=_sc_mesh(),
        out_type=out_type,
        scratch_types=scratch,
        compiler_params=pltpu.CompilerParams(use_tc_tiling_on_sc=False),
    )


def _unpack(v):
    xi = lax.bitcast_convert_type(v, jnp.int32)
    hf = lax.bitcast_convert_type(jnp.left_shift(xi, 16), jnp.float32)
    pf = lax.bitcast_convert_type(
        jnp.bitwise_and(xi, jnp.int32(-65536)), jnp.float32)
    return hf, pf


def _edge_half(hv, dv, ea, whd, whs, wsq, wea, be1, we2, be2, wxm, bx,
               with_w):
    sh, sp = _unpack(hv)
    dh, dp = _unpack(dv)
    diffp = dp - sp
    a = jnp.dot(dh, whd, preferred_element_type=jnp.float32)
    a = a + jnp.dot(sh, whs, preferred_element_type=jnp.float32)
    a = a + jnp.dot(diffp * diffp, wsq, preferred_element_type=jnp.float32)
    a = a + jnp.dot(ea, wea, preferred_element_type=jnp.float32)
    a = a + be1
    m1 = a * jax.nn.sigmoid(a)
    b = jnp.dot(m1, we2, preferred_element_type=jnp.float32) + be2
    m2 = b * jax.nn.sigmoid(b)
    if not with_w:
        return m2, None
    wpre = jnp.dot(m2, wxm, preferred_element_type=jnp.float32)
    wv = jnp.tanh(wpre[:, :1] + bx[:, :1])
    wd = diffp[:, :16] * wv
    col16 = lax.broadcasted_iota(jnp.int32, wd.shape, 1)
    wd = jnp.where(col16 == 3, 1.0, wd)
    wd3 = wd.reshape(BE // 16, 8, 16)
    wdp = jnp.concatenate([wd3[:, q, :] for q in range(8)], axis=1)
    return m2, wdp


def _edge_body(with_w, hps, hpd, eapk,
               whd, whs, wsq, wea, be1, we2, be2, wxm, bx, *outs):
    sv = hps[...]
    dv = hpd[...]
    t = eapk[...]
    eaw = jnp.stack([t[:, 16 * q:16 * q + 16] for q in range(8)],
                    axis=1).reshape(BE // 2, 16)
    eae, eao = _unpack(eaw)
    args = (whd[...], whs[...], wsq[...], wea[...], be1[...],
            we2[...], be2[...], wxm[...], bx[...])
    m2e, wde = _edge_half(sv[:, :DT], dv[:, :DT], eae, *args, with_w)
    m2o, wdo = _edge_half(sv[:, DT:], dv[:, DT:], eao, *args, with_w)
    outs[0][...] = m2e
    outs[1][...] = m2o
    if with_w:
        outs[2][...] = wde
        outs[3][...] = wdo


def _edge_call(with_w):
    w_specs = [
        pl.BlockSpec((D, H), lambda i: (0, 0)),
        pl.BlockSpec((D, H), lambda i: (0, 0)),
        pl.BlockSpec((D, H), lambda i: (0, 0)),
        pl.BlockSpec((16, H), lambda i: (0, 0)),
        pl.BlockSpec((1, H), lambda i: (0, 0)),
        pl.BlockSpec((H, H), lambda i: (0, 0)),
        pl.BlockSpec((1, H), lambda i: (0, 0)),
        pl.BlockSpec((H, H), lambda i: (0, 0)),
        pl.BlockSpec((1, H), lambda i: (0, 0)),
    ]
    out_shape = [jax.ShapeDtypeStruct((E // 2, H), jnp.float32),
                 jax.ShapeDtypeStruct((E // 2, H), jnp.float32)]
    out_specs = [pl.BlockSpec((BE // 2, H), lambda i: (i, 0)),
                 pl.BlockSpec((BE // 2, H), lambda i: (i, 0))]
    if with_w:
        out_shape += [jax.ShapeDtypeStruct((E * 8 // 128, 128), jnp.float32)] * 2
        out_specs += [pl.BlockSpec((BE // 16, 128), lambda i: (i, 0))] * 2
    return pl.pallas_call(
        functools.partial(_edge_body, with_w),
        grid=(GE,),
        in_specs=[
            pl.BlockSpec((BE // 2, 2 * DT), lambda i: (i, 0)),
            pl.BlockSpec((BE // 2, 2 * DT), lambda i: (i, 0)),
            pl.BlockSpec((BE // 16, 128), lambda i: (i, 0)),
        ] + w_specs,
        out_specs=out_specs,
        out_shape=out_shape,
        compiler_params=pltpu.CompilerParams(
            dimension_semantics=("arbitrary",)),
    )


def _node_body(with_c, xb, m0, m1, a0, a1, pp,
               wh1a, wh1b, bh1, wh2, bh2, *outs):
    xv = xb[...]
    magg = m0[...] + m1[...]
    t = (jnp.dot(xv, wh1a[...], preferred_element_type=jnp.float32)
         + jnp.dot(magg, wh1b[...], preferred_element_type=jnp.float32)
         + bh1[...])
    t = t * jax.nn.sigmoid(t)
    hn = xv + jnp.dot(t, wh2[...], preferred_element_type=jnp.float32) + bh2[...]
    if with_c:
        ax = a0[...] + a1[...]
        cnt = jnp.maximum(ax[:, 3:4], 1.0)
        upd = ax / cnt
        col = lax.broadcasted_iota(jnp.int32, upd.shape, 1)
        cn = pp[...] + jnp.where(col < 3, upd, 0.0)
        cn64 = jnp.concatenate(
            [cn, jnp.zeros((hn.shape[0], D - 16), jnp.float32)], axis=1)
        h32 = lax.bitcast_convert_type(
            hn.astype(jnp.bfloat16).astype(jnp.float32), jnp.int32)
        p32 = lax.bitcast_convert_type(
            cn64.astype(jnp.bfloat16).astype(jnp.float32), jnp.int32)
        w32 = jnp.bitwise_or(lax.shift_right_logical(h32, 16), p32)
        outs[0][...] = lax.bitcast_convert_type(w32, jnp.float32)
        outs[1][...] = hn
    else:
        outs[0][...] = hn


def _node_call(with_c):
    in_specs = [
        pl.BlockSpec((BN, D), lambda i: (i, 0)),
        pl.BlockSpec((BN, H), lambda i: (i, 0)),
        pl.BlockSpec((BN, H), lambda i: (i, 0)),
        pl.BlockSpec((BN, 16), lambda i: (i, 0)),
        pl.BlockSpec((BN, 16), lambda i: (i, 0)),
        pl.BlockSpec((BN, 16), lambda i: (i, 0)),
        pl.BlockSpec((D, H), lambda i: (0, 0)),
        pl.BlockSpec((H, H), lambda i: (0, 0)),
        pl.BlockSpec((1, H), lambda i: (0, 0)),
        pl.BlockSpec((H, D), lambda i: (0, 0)),
        pl.BlockSpec((1, D), lambda i: (0, 0)),
    ]
    if with_c:
        out_shape = [jax.ShapeDtypeStruct((N, DT), jnp.float32),
                     jax.ShapeDtypeStruct((N, D), jnp.float32)]
        out_specs = [pl.BlockSpec((BN, DT), lambda i: (i, 0)),
                     pl.BlockSpec((BN, D), lambda i: (i, 0))]
    else:
        out_shape = [jax.ShapeDtypeStruct((N, D), jnp.float32)]
        out_specs = [pl.BlockSpec((BN, D), lambda i: (i, 0))]
    return pl.pallas_call(
        functools.partial(_node_body, with_c),
        grid=(GN,),
        in_specs=in_specs,
        out_specs=out_specs,
        out_shape=out_shape,
        compiler_params=pltpu.CompilerParams(
            dimension_semantics=("arbitrary",)),
    )


def _layer_weights(lp):
    we1 = lp["We1"]
    wd2 = we1[2 * D:2 * D + 1]
    wsq = jnp.concatenate(
        [jnp.broadcast_to(wd2, (16, H)),
         jnp.zeros((D - 16, H), jnp.float32)], axis=0)
    wxm = jnp.pad(lp["Wx"], ((0, 0), (0, H - 1)))
    return dict(
        whd=we1[0:D],
        whs=we1[D:2 * D],
        wsq=wsq,
        wea=jnp.pad(we1[2 * D + 1:], ((0, 1), (0, 0))),
        be1=lp["be1"][None, :],
        we2=lp["We2"],
        be2=lp["be2"][None, :],
        wxm=wxm,
        bx=jnp.broadcast_to(lp["bx"].reshape(1, 1), (1, H)),
        wh1a=lp["Wh1"][0:D],
        wh1b=lp["Wh1"][D:],
        bh1=lp["bh1"][None, :],
        wh2=lp["Wh2"],
        bh2=lp["bh2"][None, :],
    )


def kernel(x, pos, edge_index, edge_attr, params):
    src = edge_index[0]
    dst = edge_index[1]
    src3d = src.reshape(NW, NCH, CH)
    dst3d = dst.reshape(NW, NCH, CH)
    pos_pad = jnp.pad(pos, ((0, 0), (0, 13)))
    pos64 = jnp.pad(pos, ((0, 0), (0, D - 3)))
    h32 = lax.bitcast_convert_type(
        x.astype(jnp.bfloat16).astype(jnp.float32), jnp.int32)
    p32 = lax.bitcast_convert_type(
        pos64.astype(jnp.bfloat16).astype(jnp.float32), jnp.int32)
    tab0 = lax.bitcast_convert_type(
        jnp.bitwise_or(lax.shift_right_logical(h32, 16), p32), jnp.float32)
    zm = jnp.zeros((RPT, H), jnp.float32)
    za = jnp.zeros((RPT, 16), jnp.float32)
    w1 = _layer_weights(params["layers"][0])
    w2 = _layer_weights(params["layers"][1])

    dste3d = dst[:E // 2].reshape(NW, NCH // 2, CH)
    dsto3d = dst[E // 2:].reshape(NW, NCH // 2, CH)

    ea16e = jnp.pad(edge_attr[:E // 2], ((0, 0), (0, 1)))
    ea16o = jnp.pad(edge_attr[E // 2:], ((0, 0), (0, 1)))
    el = lax.bitcast_convert_type(
        ea16e.astype(jnp.bfloat16).astype(jnp.float32), jnp.int32)
    eh = lax.bitcast_convert_type(
        ea16o.astype(jnp.bfloat16).astype(jnp.float32), jnp.int32)
    eapk = lax.bitcast_convert_type(
        jnp.bitwise_or(lax.shift_right_logical(el, 16), eh),
        jnp.float32).reshape(E * 16 // 2 // 128, 128)

    def layer(tab, wts, with_w):
        hps, hpd = _sc_gather()(tab, src3d, dst3d)
        outs = _edge_call(with_w)(
            hps, hpd, eapk,
            wts["whd"], wts["whs"], wts["wsq"], wts["wea"], wts["be1"],
            wts["we2"], wts["be2"], wts["wxm"], wts["bx"])
        if with_w:
            m2e, m2o, wde, wdo = outs
            return _sc_scatter(True)(
                m2e, m2o, wde.reshape(E // 2, 16), wdo.reshape(E // 2, 16),
                dste3d, dsto3d, zm, za)
        m2e, m2o = outs
        return _sc_scatter(False)(m2e, m2o, dste3d, dsto3d, zm)

    mparts, aparts = layer(tab0, w1, True)
    tab1, h1 = _node_call(True)(
        x, mparts[0], mparts[1], aparts[0], aparts[1], pos_pad,
        w1["wh1a"], w1["wh1b"], w1["bh1"], w1["wh2"], w1["bh2"])

    mparts2, = layer(tab1, w2, False)
    h2, = _node_call(False)(
        h1, mparts2[0], mparts2[1], aparts[0], aparts[1], pos_pad,
        w2["wh1a"], w2["wh1b"], w2["bh1"], w2["wh2"], w2["bh2"])
    return h2


# speedup vs baseline: 1.0106x; 1.0106x over previous
import functools

import jax
import jax.numpy as jnp
from jax import lax
from jax.experimental import pallas as pl
from jax.experimental.pallas import tpu as pltpu
from jax.experimental.pallas import tpu_sc as plsc

N = 10000
E = 640000
D = 64
DT = 64
H = 128
DE = 15
NC = 2
NS = 16
NW = NC * NS
EPW = E // NW
CH = 80
NCH = EPW // CH
NP = 10240
RPT = NP // NS
BE = 1024
GE = E // BE
BN = 1000
GN = N // BN


def _sc_gather_body(tab_hbm, src_hbm, dst_hbm, hps_out, hpd_out,
                    idx_s, idx_d, bs0, bd0, bs1, bd1, sem0, sem1):
    c = lax.axis_index("c")
    s = lax.axis_index("s")
    wid = s * NC + c
    half = wid // (NW // 2)
    widh = wid % (NW // 2)
    lane0 = half * DT
    pltpu.sync_copy(src_hbm.at[wid], idx_s)
    pltpu.sync_copy(dst_hbm.at[wid], idx_d)
    bufs = ((bs0, bd0, sem0), (bs1, bd1, sem1))

    def start(g, b):
        bs, bd, sem = bufs[b]
        pltpu.make_async_copy(tab_hbm.at[idx_s.at[g]], bs, sem).start()
        pltpu.make_async_copy(tab_hbm.at[idx_d.at[g]], bd, sem).start()

    def finish(g, b):
        bs, bd, sem = bufs[b]
        pltpu.make_async_copy(tab_hbm.at[idx_s.at[g]], bs, sem).wait()
        pltpu.make_async_copy(tab_hbm.at[idx_d.at[g]], bd, sem).wait()
        r0 = widh * EPW + g * CH
        pltpu.sync_copy(bs, hps_out.at[pl.ds(r0, CH), pl.ds(lane0, DT)])
        pltpu.sync_copy(bd, hpd_out.at[pl.ds(r0, CH), pl.ds(lane0, DT)])

    start(0, 0)

    def body(i, carry):
        g0 = 2 * i
        start(g0 + 1, 1)
        finish(g0, 0)

        @pl.when(i < NCH // 2 - 1)
        def _():
            start(g0 + 2, 0)

        finish(g0 + 1, 1)
        return carry

    lax.fori_loop(0, NCH // 2, body, 0)


def _sc_scatter_body(with_w, *refs):
    if with_w:
        (m2e_hbm, m2o_hbm, wde_hbm, wdo_hbm, dste_hbm, dsto_hbm,
         zm_hbm, za_hbm, outm, outa,
         idx0, idx1, m0, m1, w0, w1, macc, aacc, sem0, sem1) = refs
        bufs = ((idx0, m0, w0, sem0), (idx1, m1, w1, sem1))
    else:
        (m2e_hbm, m2o_hbm, dste_hbm, dsto_hbm, zm_hbm, outm,
         idx0, idx1, m0, m1, macc, sem0, sem1) = refs
        wde_hbm = wdo_hbm = None
        bufs = ((idx0, m0, None, sem0), (idx1, m1, None, sem1))
    c = lax.axis_index("c")
    s = lax.axis_index("s")
    wid = s * NC + c
    pltpu.sync_copy(zm_hbm, macc.at[pl.ds(s * RPT, RPT)])
    if with_w:
        pltpu.sync_copy(za_hbm, aacc.at[pl.ds(s * RPT, RPT)])
    plsc.subcore_barrier()
    epw_h = EPW // 2

    def run_pass(m2_hbm, wd_hbm, dst_hbm):
        def start(g, b):
            iv, mv, wv, sem = bufs[b]
            e0 = wid * epw_h + g * CH
            pltpu.make_async_copy(dst_hbm.at[wid].at[pl.ds(g, 1)],
                                  iv, sem).start()
            pltpu.make_async_copy(m2_hbm.at[pl.ds(e0, CH)], mv, sem).start()
            if with_w:
                pltpu.make_async_copy(wd_hbm.at[pl.ds(e0, CH)],
                                      wv, sem).start()

        def finish(g, b):
            iv, mv, wv, sem = bufs[b]
            e0 = wid * epw_h + g * CH
            pltpu.make_async_copy(dst_hbm.at[wid].at[pl.ds(g, 1)],
                                  iv, sem).wait()
            pltpu.make_async_copy(m2_hbm.at[pl.ds(e0, CH)], mv, sem).wait()
            if with_w:
                pltpu.make_async_copy(wd_hbm.at[pl.ds(e0, CH)],
                                      wv, sem).wait()
            pltpu.sync_copy(mv, macc.at[iv.at[0]], add=True)
            if with_w:
                pltpu.sync_copy(wv, aacc.at[iv.at[0]], add=True)

        nch_h = NCH // 2
        start(0, 0)

        def body(i, carry):
            g0 = 2 * i

            @pl.when(g0 + 1 < nch_h)
            def _():
                start(g0 + 1, 1)

            finish(g0, 0)

            @pl.when(g0 + 2 < nch_h)
            def _():
                start(g0 + 2, 0)

            @pl.when(g0 + 1 < nch_h)
            def _():
                finish(g0 + 1, 1)

            return carry

        lax.fori_loop(0, (nch_h + 1) // 2, body, 0)

    run_pass(m2e_hbm, wde_hbm, dste_hbm)
    run_pass(m2o_hbm, wdo_hbm, dsto_hbm)
    plsc.subcore_barrier()
    pltpu.sync_copy(macc.at[pl.ds(s * RPT, RPT)],
                    outm.at[c].at[pl.ds(s * RPT, RPT)])
    if with_w:
        pltpu.sync_copy(aacc.at[pl.ds(s * RPT, RPT)],
                        outa.at[c].at[pl.ds(s * RPT, RPT)])


@functools.lru_cache(maxsize=None)
def _sc_mesh():
    return plsc.VectorSubcoreMesh(core_axis_name="c", subcore_axis_name="s")


@functools.lru_cache(maxsize=None)
def _sc_gather():
    return pl.kernel(
        _sc_gather_body,
        mesh=_sc_mesh(),
        out_type=[
            jax.ShapeDtypeStruct((E // 2, 2 * DT), jnp.float32),
            jax.ShapeDtypeStruct((E // 2, 2 * DT), jnp.float32),
        ],
        scratch_types=[
            pltpu.VMEM((NCH, CH), jnp.int32),
            pltpu.VMEM((NCH, CH), jnp.int32),
            pltpu.VMEM((CH, DT), jnp.float32),
            pltpu.VMEM((CH, DT), jnp.float32),
            pltpu.VMEM((CH, DT), jnp.float32),
            pltpu.VMEM((CH, DT), jnp.float32),
            pltpu.SemaphoreType.DMA,
            pltpu.SemaphoreType.DMA,
        ],
        compiler_params=pltpu.CompilerParams(use_tc_tiling_on_sc=False),
    )


@functools.lru_cache(maxsize=None)
def _sc_scatter(with_w):
    out_type = [jax.ShapeDtypeStruct((NC, NP, H), jnp.float32)]
    scratch = [
        pltpu.VMEM((1, CH), jnp.int32),
        pltpu.VMEM((1, CH), jnp.int32),
        pltpu.VMEM((CH, H), jnp.float32),
        pltpu.VMEM((CH, H), jnp.float32),
    ]
    if with_w:
        out_type.append(jax.ShapeDtypeStruct((NC, NP, 16), jnp.float32))
        scratch.append(pltpu.VMEM((CH, 16), jnp.float32))
        scratch.append(pltpu.VMEM((CH, 16), jnp.float32))
    scratch.append(pltpu.VMEM_SHARED((NP, H), jnp.float32))
    if with_w:
        scratch.append(pltpu.VMEM_SHARED((NP, 16), jnp.float32))
    scratch.append(pltpu.SemaphoreType.DMA)
    scratch.append(pltpu.SemaphoreType.DMA)
    return pl.kernel(
        functools.partial(_sc_scatter_body, with_w),
        mesh=_sc_mesh(),
        out_type=out_type,
        scratch_types=scratch,
        compiler_params=pltpu.CompilerParams(use_tc_tiling_on_sc=False),
    )


def _unpack(v):
    xi = lax.bitcast_convert_type(v, jnp.int32)
    hf = lax.bitcast_convert_type(jnp.left_shift(xi, 16), jnp.float32)
    pf = lax.bitcast_convert_type(
        jnp.bitwise_and(xi, jnp.int32(-65536)), jnp.float32)
    return hf, pf


def _edge_half(hv, dv, ea, whd, whs, wsq, wea, be1, we2, be2, wxm, bx,
               with_w):
    sh, sp = _unpack(hv)
    dh, dp = _unpack(dv)
    diffp = dp - sp
    a = jnp.dot(dh, whd, preferred_element_type=jnp.float32)
    a = a + jnp.dot(sh, whs, preferred_element_type=jnp.float32)
    a = a + jnp.dot(diffp * diffp, wsq, preferred_element_type=jnp.float32)
    a = a + jnp.dot(ea, wea, preferred_element_type=jnp.float32)
    a = a + be1
    m1 = a * jax.nn.sigmoid(a)
    b = jnp.dot(m1, we2, preferred_element_type=jnp.float32) + be2
    m2 = b * jax.nn.sigmoid(b)
    if not with_w:
        return m2, None
    wpre = jnp.dot(m2, wxm, preferred_element_type=jnp.float32)
    wv = jnp.tanh(wpre[:, :1] + bx[:, :1])
    wd = diffp[:, :16] * wv
    col16 = lax.broadcasted_iota(jnp.int32, wd.shape, 1)
    wd = jnp.where(col16 == 3, 1.0, wd)
    wd3 = wd.reshape(BE // 16, 8, 16)
    wdp = jnp.concatenate([wd3[:, q, :] for q in range(8)], axis=1)
    return m2, wdp


def _edge_body(with_w, hps, hpd, eap,
               whd, whs, wsq, wea, be1, we2, be2, wxm, bx, *outs):
    sv = hps[...]
    dv = hpd[...]
    eav = eap[...]
    args = (whd[...], whs[...], wsq[...], wea[...], be1[...],
            we2[...], be2[...], wxm[...], bx[...])
    m2e, wde = _edge_half(sv[:, :DT], dv[:, :DT], eav[:, :DE], *args, with_w)
    m2o, wdo = _edge_half(sv[:, DT:], dv[:, DT:], eav[:, DE:2 * DE],
                          *args, with_w)
    outs[0][...] = m2e
    outs[1][...] = m2o
    if with_w:
        outs[2][...] = wde
        outs[3][...] = wdo


def _edge_call(with_w):
    w_specs = [
        pl.BlockSpec((D, H), lambda i: (0, 0)),
        pl.BlockSpec((D, H), lambda i: (0, 0)),
        pl.BlockSpec((D, H), lambda i: (0, 0)),
        pl.BlockSpec((DE, H), lambda i: (0, 0)),
        pl.BlockSpec((1, H), lambda i: (0, 0)),
        pl.BlockSpec((H, H), lambda i: (0, 0)),
        pl.BlockSpec((1, H), lambda i: (0, 0)),
        pl.BlockSpec((H, H), lambda i: (0, 0)),
        pl.BlockSpec((1, H), lambda i: (0, 0)),
    ]
    out_shape = [jax.ShapeDtypeStruct((E // 2, H), jnp.float32),
                 jax.ShapeDtypeStruct((E // 2, H), jnp.float32)]
    out_specs = [pl.BlockSpec((BE // 2, H), lambda i: (i, 0)),
                 pl.BlockSpec((BE // 2, H), lambda i: (i, 0))]
    if with_w:
        out_shape += [jax.ShapeDtypeStruct((E * 8 // 128, 128), jnp.float32)] * 2
        out_specs += [pl.BlockSpec((BE // 16, 128), lambda i: (i, 0))] * 2
    return pl.pallas_call(
        functools.partial(_edge_body, with_w),
        grid=(GE,),
        in_specs=[
            pl.BlockSpec((BE // 2, 2 * DT), lambda i: (i, 0)),
            pl.BlockSpec((BE // 2, 2 * DT), lambda i: (i, 0)),
            pl.BlockSpec((BE // 2, 128), lambda i: (i, 0)),
        ] + w_specs,
        out_specs=out_specs,
        out_shape=out_shape,
        compiler_params=pltpu.CompilerParams(
            dimension_semantics=("arbitrary",)),
    )


def _node_body(with_c, xb, m0, m1, a0, a1, pp,
               wh1a, wh1b, bh1, wh2, bh2, *outs):
    xv = xb[...]
    magg = m0[...] + m1[...]
    t = (jnp.dot(xv, wh1a[...], preferred_element_type=jnp.float32)
         + jnp.dot(magg, wh1b[...], preferred_element_type=jnp.float32)
         + bh1[...])
    t = t * jax.nn.sigmoid(t)
    hn = xv + jnp.dot(t, wh2[...], preferred_element_type=jnp.float32) + bh2[...]
    if with_c:
        ax = a0[...] + a1[...]
        cnt = jnp.maximum(ax[:, 3:4], 1.0)
        upd = ax / cnt
        col = lax.broadcasted_iota(jnp.int32, upd.shape, 1)
        cn = pp[...] + jnp.where(col < 3, upd, 0.0)
        cn64 = jnp.concatenate(
            [cn, jnp.zeros((hn.shape[0], D - 16), jnp.float32)], axis=1)
        h32 = lax.bitcast_convert_type(
            hn.astype(jnp.bfloat16).astype(jnp.float32), jnp.int32)
        p32 = lax.bitcast_convert_type(
            cn64.astype(jnp.bfloat16).astype(jnp.float32), jnp.int32)
        w32 = jnp.bitwise_or(lax.shift_right_logical(h32, 16), p32)
        outs[0][...] = lax.bitcast_convert_type(w32, jnp.float32)
        outs[1][...] = hn
    else:
        outs[0][...] = hn


def _node_call(with_c):
    in_specs = [
        pl.BlockSpec((BN, D), lambda i: (i, 0)),
        pl.BlockSpec((BN, H), lambda i: (i, 0)),
        pl.BlockSpec((BN, H), lambda i: (i, 0)),
        pl.BlockSpec((BN, 16), lambda i: (i, 0)),
        pl.BlockSpec((BN, 16), lambda i: (i, 0)),
        pl.BlockSpec((BN, 16), lambda i: (i, 0)),
        pl.BlockSpec((D, H), lambda i: (0, 0)),
        pl.BlockSpec((H, H), lambda i: (0, 0)),
        pl.BlockSpec((1, H), lambda i: (0, 0)),
        pl.BlockSpec((H, D), lambda i: (0, 0)),
        pl.BlockSpec((1, D), lambda i: (0, 0)),
    ]
    if with_c:
        out_shape = [jax.ShapeDtypeStruct((N, DT), jnp.float32),
                     jax.ShapeDtypeStruct((N, D), jnp.float32)]
        out_specs = [pl.BlockSpec((BN, DT), lambda i: (i, 0)),
                     pl.BlockSpec((BN, D), lambda i: (i, 0))]
    else:
        out_shape = [jax.ShapeDtypeStruct((N, D), jnp.float32)]
        out_specs = [pl.BlockSpec((BN, D), lambda i: (i, 0))]
    return pl.pallas_call(
        functools.partial(_node_body, with_c),
        grid=(GN,),
        in_specs=in_specs,
        out_specs=out_specs,
        out_shape=out_shape,
        compiler_params=pltpu.CompilerParams(
            dimension_semantics=("arbitrary",)),
    )


def _layer_weights(lp):
    we1 = lp["We1"]
    wd2 = we1[2 * D:2 * D + 1]
    wsq = jnp.concatenate(
        [jnp.broadcast_to(wd2, (16, H)),
         jnp.zeros((D - 16, H), jnp.float32)], axis=0)
    wxm = jnp.pad(lp["Wx"], ((0, 0), (0, H - 1)))
    return dict(
        whd=we1[0:D],
        whs=we1[D:2 * D],
        wsq=wsq,
        wea=we1[2 * D + 1:],
        be1=lp["be1"][None, :],
        we2=lp["We2"],
        be2=lp["be2"][None, :],
        wxm=wxm,
        bx=jnp.broadcast_to(lp["bx"].reshape(1, 1), (1, H)),
        wh1a=lp["Wh1"][0:D],
        wh1b=lp["Wh1"][D:],
        bh1=lp["bh1"][None, :],
        wh2=lp["Wh2"],
        bh2=lp["bh2"][None, :],
    )


def kernel(x, pos, edge_index, edge_attr, params):
    src = edge_index[0]
    dst = edge_index[1]
    src3d = src.reshape(NW, NCH, CH)
    dst3d = dst.reshape(NW, NCH, CH)
    pos_pad = jnp.pad(pos, ((0, 0), (0, 13)))
    pos64 = jnp.pad(pos, ((0, 0), (0, D - 3)))
    h32 = lax.bitcast_convert_type(
        x.astype(jnp.bfloat16).astype(jnp.float32), jnp.int32)
    p32 = lax.bitcast_convert_type(
        pos64.astype(jnp.bfloat16).astype(jnp.float32), jnp.int32)
    tab0 = lax.bitcast_convert_type(
        jnp.bitwise_or(lax.shift_right_logical(h32, 16), p32), jnp.float32)
    zm = jnp.zeros((RPT, H), jnp.float32)
    za = jnp.zeros((RPT, 16), jnp.float32)
    w1 = _layer_weights(params["layers"][0])
    w2 = _layer_weights(params["layers"][1])

    dste3d = dst[:E // 2].reshape(NW, NCH // 2, CH)
    dsto3d = dst[E // 2:].reshape(NW, NCH // 2, CH)

    eaP = jnp.concatenate(
        [edge_attr[:E // 2], edge_attr[E // 2:],
         jnp.zeros((E // 2, 128 - 2 * DE), jnp.float32)], axis=1)

    def layer(tab, wts, with_w):
        hps, hpd = _sc_gather()(tab, src3d, dst3d)
        outs = _edge_call(with_w)(
            hps, hpd, eaP,
            wts["whd"], wts["whs"], wts["wsq"], wts["wea"], wts["be1"],
            wts["we2"], wts["be2"], wts["wxm"], wts["bx"])
        if with_w:
            m2e, m2o, wde, wdo = outs
            return _sc_scatter(True)(
                m2e, m2o, wde.reshape(E // 2, 16), wdo.reshape(E // 2, 16),
                dste3d, dsto3d, zm, za)
        m2e, m2o = outs
        return _sc_scatter(False)(m2e, m2o, dste3d, dsto3d, zm)

    mparts, aparts = layer(tab0, w1, True)
    tab1, h1 = _node_call(True)(
        x, mparts[0], mparts[1], aparts[0], aparts[1], pos_pad,
        w1["wh1a"], w1["wh1b"], w1["bh1"], w1["wh2"], w1["bh2"])

    mparts2, = layer(tab1, w2, False)
    h2, = _node_call(False)(
        h1, mparts2[0], mparts2[1], aparts[0], aparts[1], pos_pad,
        w2["wh1a"], w2["wh1b"], w2["bh1"], w2["wh2"], w2["bh2"])
    return h2


# speedup vs baseline: 1.0591x; 1.0480x over previous
import functools

import jax
import jax.numpy as jnp
from jax import lax
from jax.experimental import pallas as pl
from jax.experimental.pallas import tpu as pltpu
from jax.experimental.pallas import tpu_sc as plsc

N = 10000
E = 640000
D = 64
DT = 64
H = 128
DE = 15
NC = 2
NS = 16
NW = NC * NS
EPW = E // NW
CH = 80
NCH = EPW // CH
NP = 10240
RPT = NP // NS
BE = 1024
GE = E // BE
BN = 1000
GN = N // BN


def _sc_gather_body(tab_hbm, src_hbm, dst_hbm, hps_out, hpd_out,
                    idx_s, idx_d, bs0, bd0, bs1, bd1, sem0, sem1):
    c = lax.axis_index("c")
    s = lax.axis_index("s")
    wid = s * NC + c
    half = wid // (NW // 2)
    widh = wid % (NW // 2)
    lane0 = half * DT
    pltpu.sync_copy(src_hbm.at[wid], idx_s)
    pltpu.sync_copy(dst_hbm.at[wid], idx_d)
    bufs = ((bs0, bd0, sem0), (bs1, bd1, sem1))

    def start(g, b):
        bs, bd, sem = bufs[b]
        pltpu.make_async_copy(tab_hbm.at[idx_s.at[g]], bs, sem).start()
        pltpu.make_async_copy(tab_hbm.at[idx_d.at[g]], bd, sem).start()

    def finish(g, b):
        bs, bd, sem = bufs[b]
        pltpu.make_async_copy(tab_hbm.at[idx_s.at[g]], bs, sem).wait()
        pltpu.make_async_copy(tab_hbm.at[idx_d.at[g]], bd, sem).wait()
        r0 = widh * EPW + g * CH
        pltpu.sync_copy(bs, hps_out.at[pl.ds(r0, CH), pl.ds(lane0, DT)])
        pltpu.sync_copy(bd, hpd_out.at[pl.ds(r0, CH), pl.ds(lane0, DT)])

    start(0, 0)

    def body(i, carry):
        g0 = 2 * i
        start(g0 + 1, 1)
        finish(g0, 0)

        @pl.when(i < NCH // 2 - 1)
        def _():
            start(g0 + 2, 0)

        finish(g0 + 1, 1)
        return carry

    lax.fori_loop(0, NCH // 2, body, 0)


def _sc_scatter_body(with_w, *refs):
    if with_w:
        (m2e_hbm, m2o_hbm, wde_hbm, wdo_hbm, dste_hbm, dsto_hbm,
         zm_hbm, za_hbm, outm, outa,
         idx0, idx1, m0, m1, w0, w1, macc, aacc, sem0, sem1) = refs
        bufs = ((idx0, m0, w0, sem0), (idx1, m1, w1, sem1))
    else:
        (m2e_hbm, m2o_hbm, dste_hbm, dsto_hbm, zm_hbm, outm,
         idx0, idx1, m0, m1, macc, sem0, sem1) = refs
        wde_hbm = wdo_hbm = None
        bufs = ((idx0, m0, None, sem0), (idx1, m1, None, sem1))
    c = lax.axis_index("c")
    s = lax.axis_index("s")
    wid = s * NC + c
    pltpu.sync_copy(zm_hbm, macc.at[pl.ds(s * RPT, RPT)])
    if with_w:
        pltpu.sync_copy(za_hbm, aacc.at[pl.ds(s * RPT, RPT)])
    plsc.subcore_barrier()
    epw_h = EPW // 2

    def run_pass(m2_hbm, wd_hbm, dst_hbm):
        def start(g, b):
            iv, mv, wv, sem = bufs[b]
            e0 = wid * epw_h + g * CH
            pltpu.make_async_copy(dst_hbm.at[wid].at[pl.ds(g, 1)],
                                  iv, sem).start()
            pltpu.make_async_copy(m2_hbm.at[pl.ds(e0, CH)], mv, sem).start()
            if with_w:
                pltpu.make_async_copy(wd_hbm.at[pl.ds(e0, CH)],
                                      wv, sem).start()

        def finish(g, b):
            iv, mv, wv, sem = bufs[b]
            e0 = wid * epw_h + g * CH
            pltpu.make_async_copy(dst_hbm.at[wid].at[pl.ds(g, 1)],
                                  iv, sem).wait()
            pltpu.make_async_copy(m2_hbm.at[pl.ds(e0, CH)], mv, sem).wait()
            if with_w:
                pltpu.make_async_copy(wd_hbm.at[pl.ds(e0, CH)],
                                      wv, sem).wait()
            pltpu.sync_copy(mv, macc.at[iv.at[0]], add=True)
            if with_w:
                pltpu.sync_copy(wv, aacc.at[iv.at[0]], add=True)

        nch_h = NCH // 2
        start(0, 0)

        def body(i, carry):
            g0 = 2 * i

            @pl.when(g0 + 1 < nch_h)
            def _():
                start(g0 + 1, 1)

            finish(g0, 0)

            @pl.when(g0 + 2 < nch_h)
            def _():
                start(g0 + 2, 0)

            @pl.when(g0 + 1 < nch_h)
            def _():
                finish(g0 + 1, 1)

            return carry

        lax.fori_loop(0, (nch_h + 1) // 2, body, 0)

    run_pass(m2e_hbm, wde_hbm, dste_hbm)
    run_pass(m2o_hbm, wdo_hbm, dsto_hbm)
    plsc.subcore_barrier()
    pltpu.sync_copy(macc.at[pl.ds(s * RPT, RPT)],
                    outm.at[c].at[pl.ds(s * RPT, RPT)])
    if with_w:
        pltpu.sync_copy(aacc.at[pl.ds(s * RPT, RPT)],
                        outa.at[c].at[pl.ds(s * RPT, RPT)])


@functools.lru_cache(maxsize=None)
def _sc_mesh():
    return plsc.VectorSubcoreMesh(core_axis_name="c", subcore_axis_name="s")


@functools.lru_cache(maxsize=None)
def _sc_gather():
    return pl.kernel(
        _sc_gather_body,
        mesh=_sc_mesh(),
        out_type=[
            jax.ShapeDtypeStruct((E // 2, 2 * DT), jnp.float32),
            jax.ShapeDtypeStruct((E // 2, 2 * DT), jnp.float32),
        ],
        scratch_types=[
            pltpu.VMEM((NCH, CH), jnp.int32),
            pltpu.VMEM((NCH, CH), jnp.int32),
            pltpu.VMEM((CH, DT), jnp.float32),
            pltpu.VMEM((CH, DT), jnp.float32),
            pltpu.VMEM((CH, DT), jnp.float32),
            pltpu.VMEM((CH, DT), jnp.float32),
            pltpu.SemaphoreType.DMA,
            pltpu.SemaphoreType.DMA,
        ],
        compiler_params=pltpu.CompilerParams(use_tc_tiling_on_sc=False),
    )


@functools.lru_cache(maxsize=None)
def _sc_scatter(with_w):
    out_type = [jax.ShapeDtypeStruct((NC, NP, H), jnp.float32)]
    scratch = [
        pltpu.VMEM((1, CH), jnp.int32),
        pltpu.VMEM((1, CH), jnp.int32),
        pltpu.VMEM((CH, H), jnp.float32),
        pltpu.VMEM((CH, H), jnp.float32),
    ]
    if with_w:
        out_type.append(jax.ShapeDtypeStruct((NC, NP, 16), jnp.float32))
        scratch.append(pltpu.VMEM((CH, 16), jnp.float32))
        scratch.append(pltpu.VMEM((CH, 16), jnp.float32))
    scratch.append(pltpu.VMEM_SHARED((NP, H), jnp.float32))
    if with_w:
        scratch.append(pltpu.VMEM_SHARED((NP, 16), jnp.float32))
    scratch.append(pltpu.SemaphoreType.DMA)
    scratch.append(pltpu.SemaphoreType.DMA)
    return pl.kernel(
        functools.partial(_sc_scatter_body, with_w),
        mesh=_sc_mesh(),
        out_type=out_type,
        scratch_types=scratch,
        compiler_params=pltpu.CompilerParams(use_tc_tiling_on_sc=False),
    )


def _unpack(v):
    xi = lax.bitcast_convert_type(v, jnp.int32)
    hf = lax.bitcast_convert_type(jnp.left_shift(xi, 16), jnp.float32)
    pf = lax.bitcast_convert_type(
        jnp.bitwise_and(xi, jnp.int32(-65536)), jnp.float32)
    return hf, pf


def _edge_half(hv, dv, ea, whd, whs, wsq, wea, be1, we2, be2, wxm, bx,
               with_w):
    sh, sp = _unpack(hv)
    dh, dp = _unpack(dv)
    diffp = dp - sp
    a = jnp.dot(dh, whd, preferred_element_type=jnp.float32)
    a = a + jnp.dot(sh, whs, preferred_element_type=jnp.float32)
    a = a + jnp.dot(diffp * diffp, wsq, preferred_element_type=jnp.float32)
    a = a + jnp.dot(ea, wea, preferred_element_type=jnp.float32)
    a = a + be1
    m1 = a * jax.nn.sigmoid(a)
    b = jnp.dot(m1, we2, preferred_element_type=jnp.float32) + be2
    m2 = b * jax.nn.sigmoid(b)
    if not with_w:
        return m2, None
    wpre = jnp.dot(m2, wxm, preferred_element_type=jnp.float32)
    wv = jnp.tanh(wpre[:, :1] + bx[:, :1])
    wd = diffp[:, :16] * wv
    col16 = lax.broadcasted_iota(jnp.int32, wd.shape, 1)
    wd = jnp.where(col16 == 3, 1.0, wd)
    wd3 = wd.reshape(BE // 16, 8, 16)
    wdp = jnp.concatenate([wd3[:, q, :] for q in range(8)], axis=1)
    return m2, wdp


def _edge_body(with_w, hps, hpd, eae, eao,
               whd, whs, wsq, wea, be1, we2, be2, wxm, bx, *outs):
    sv = hps[...]
    dv = hpd[...]
    args = (whd[...], whs[...], wsq[...], wea[...], be1[...],
            we2[...], be2[...], wxm[...], bx[...])
    m2e, wde = _edge_half(sv[:, :DT], dv[:, :DT], eae[...], *args, with_w)
    m2o, wdo = _edge_half(sv[:, DT:], dv[:, DT:], eao[...], *args, with_w)
    outs[0][...] = m2e
    outs[1][...] = m2o
    if with_w:
        outs[2][...] = wde
        outs[3][...] = wdo


def _edge_call(with_w):
    w_specs = [
        pl.BlockSpec((D, H), lambda i: (0, 0)),
        pl.BlockSpec((D, H), lambda i: (0, 0)),
        pl.BlockSpec((D, H), lambda i: (0, 0)),
        pl.BlockSpec((DE, H), lambda i: (0, 0)),
        pl.BlockSpec((1, H), lambda i: (0, 0)),
        pl.BlockSpec((H, H), lambda i: (0, 0)),
        pl.BlockSpec((1, H), lambda i: (0, 0)),
        pl.BlockSpec((H, H), lambda i: (0, 0)),
        pl.BlockSpec((1, H), lambda i: (0, 0)),
    ]
    out_shape = [jax.ShapeDtypeStruct((E // 2, H), jnp.float32),
                 jax.ShapeDtypeStruct((E // 2, H), jnp.float32)]
    out_specs = [pl.BlockSpec((BE // 2, H), lambda i: (i, 0)),
                 pl.BlockSpec((BE // 2, H), lambda i: (i, 0))]
    if with_w:
        out_shape += [jax.ShapeDtypeStruct((E * 8 // 128, 128), jnp.float32)] * 2
        out_specs += [pl.BlockSpec((BE // 16, 128), lambda i: (i, 0))] * 2
    return pl.pallas_call(
        functools.partial(_edge_body, with_w),
        grid=(GE,),
        in_specs=[
            pl.BlockSpec((BE // 2, 2 * DT), lambda i: (i, 0)),
            pl.BlockSpec((BE // 2, 2 * DT), lambda i: (i, 0)),
            pl.BlockSpec((BE // 2, DE), lambda i: (i, 0)),
            pl.BlockSpec((BE // 2, DE), lambda i: (i + GE, 0)),
        ] + w_specs,
        out_specs=out_specs,
        out_shape=out_shape,
        compiler_params=pltpu.CompilerParams(
            dimension_semantics=("arbitrary",)),
    )


def _node_body(with_c, xb, m0, m1, a0, a1, pp,
               wh1a, wh1b, bh1, wh2, bh2, *outs):
    xv = xb[...]
    magg = m0[...] + m1[...]
    t = (jnp.dot(xv, wh1a[...], preferred_element_type=jnp.float32)
         + jnp.dot(magg, wh1b[...], preferred_element_type=jnp.float32)
         + bh1[...])
    t = t * jax.nn.sigmoid(t)
    hn = xv + jnp.dot(t, wh2[...], preferred_element_type=jnp.float32) + bh2[...]
    if with_c:
        ax = a0[...] + a1[...]
        cnt = jnp.maximum(ax[:, 3:4], 1.0)
        upd = ax / cnt
        col = lax.broadcasted_iota(jnp.int32, upd.shape, 1)
        cn = pp[...] + jnp.where(col < 3, upd, 0.0)
        cn64 = jnp.concatenate(
            [cn, jnp.zeros((hn.shape[0], D - 16), jnp.float32)], axis=1)
        h32 = lax.bitcast_convert_type(
            hn.astype(jnp.bfloat16).astype(jnp.float32), jnp.int32)
        p32 = lax.bitcast_convert_type(
            cn64.astype(jnp.bfloat16).astype(jnp.float32), jnp.int32)
        w32 = jnp.bitwise_or(lax.shift_right_logical(h32, 16), p32)
        outs[0][...] = lax.bitcast_convert_type(w32, jnp.float32)
        outs[1][...] = hn
    else:
        outs[0][...] = hn


def _node_call(with_c):
    in_specs = [
        pl.BlockSpec((BN, D), lambda i: (i, 0)),
        pl.BlockSpec((BN, H), lambda i: (i, 0)),
        pl.BlockSpec((BN, H), lambda i: (i, 0)),
        pl.BlockSpec((BN, 16), lambda i: (i, 0)),
        pl.BlockSpec((BN, 16), lambda i: (i, 0)),
        pl.BlockSpec((BN, 16), lambda i: (i, 0)),
        pl.BlockSpec((D, H), lambda i: (0, 0)),
        pl.BlockSpec((H, H), lambda i: (0, 0)),
        pl.BlockSpec((1, H), lambda i: (0, 0)),
        pl.BlockSpec((H, D), lambda i: (0, 0)),
        pl.BlockSpec((1, D), lambda i: (0, 0)),
    ]
    if with_c:
        out_shape = [jax.ShapeDtypeStruct((N, DT), jnp.float32),
                     jax.ShapeDtypeStruct((N, D), jnp.float32)]
        out_specs = [pl.BlockSpec((BN, DT), lambda i: (i, 0)),
                     pl.BlockSpec((BN, D), lambda i: (i, 0))]
    else:
        out_shape = [jax.ShapeDtypeStruct((N, D), jnp.float32)]
        out_specs = [pl.BlockSpec((BN, D), lambda i: (i, 0))]
    return pl.pallas_call(
        functools.partial(_node_body, with_c),
        grid=(GN,),
        in_specs=in_specs,
        out_specs=out_specs,
        out_shape=out_shape,
        compiler_params=pltpu.CompilerParams(
            dimension_semantics=("arbitrary",)),
    )


def _layer_weights(lp):
    we1 = lp["We1"]
    wd2 = we1[2 * D:2 * D + 1]
    wsq = jnp.concatenate(
        [jnp.broadcast_to(wd2, (16, H)),
         jnp.zeros((D - 16, H), jnp.float32)], axis=0)
    wxm = jnp.pad(lp["Wx"], ((0, 0), (0, H - 1)))
    return dict(
        whd=we1[0:D],
        whs=we1[D:2 * D],
        wsq=wsq,
        wea=we1[2 * D + 1:],
        be1=lp["be1"][None, :],
        we2=lp["We2"],
        be2=lp["be2"][None, :],
        wxm=wxm,
        bx=jnp.broadcast_to(lp["bx"].reshape(1, 1), (1, H)),
        wh1a=lp["Wh1"][0:D],
        wh1b=lp["Wh1"][D:],
        bh1=lp["bh1"][None, :],
        wh2=lp["Wh2"],
        bh2=lp["bh2"][None, :],
    )


def kernel(x, pos, edge_index, edge_attr, params):
    src = edge_index[0]
    dst = edge_index[1]
    src3d = src.reshape(NW, NCH, CH)
    dst3d = dst.reshape(NW, NCH, CH)
    pos_pad = jnp.pad(pos, ((0, 0), (0, 13)))
    pos64 = jnp.pad(pos, ((0, 0), (0, D - 3)))
    h32 = lax.bitcast_convert_type(
        x.astype(jnp.bfloat16).astype(jnp.float32), jnp.int32)
    p32 = lax.bitcast_convert_type(
        pos64.astype(jnp.bfloat16).astype(jnp.float32), jnp.int32)
    tab0 = lax.bitcast_convert_type(
        jnp.bitwise_or(lax.shift_right_logical(h32, 16), p32), jnp.float32)
    zm = jnp.zeros((RPT, H), jnp.float32)
    za = jnp.zeros((RPT, 16), jnp.float32)
    w1 = _layer_weights(params["layers"][0])
    w2 = _layer_weights(params["layers"][1])

    dste3d = dst[:E // 2].reshape(NW, NCH // 2, CH)
    dsto3d = dst[E // 2:].reshape(NW, NCH // 2, CH)

    def layer(tab, wts, with_w):
        hps, hpd = _sc_gather()(tab, src3d, dst3d)
        outs = _edge_call(with_w)(
            hps, hpd, edge_attr, edge_attr,
            wts["whd"], wts["whs"], wts["wsq"], wts["wea"], wts["be1"],
            wts["we2"], wts["be2"], wts["wxm"], wts["bx"])
        if with_w:
            m2e, m2o, wde, wdo = outs
            return _sc_scatter(True)(
                m2e, m2o, wde.reshape(E // 2, 16), wdo.reshape(E // 2, 16),
                dste3d, dsto3d, zm, za)
        m2e, m2o = outs
        return _sc_scatter(False)(m2e, m2o, dste3d, dsto3d, zm)

    mparts, aparts = layer(tab0, w1, True)
    tab1, h1 = _node_call(True)(
        x, mparts[0], mparts[1], aparts[0], aparts[1], pos_pad,
        w1["wh1a"], w1["wh1b"], w1["bh1"], w1["wh2"], w1["bh2"])

    mparts2, = layer(tab1, w2, False)
    h2, = _node_call(False)(
        h1, mparts2[0], mparts2[1], aparts[0], aparts[1], pos_pad,
        w2["wh1a"], w2["wh1b"], w2["bh1"], w2["wh2"], w2["bh2"])
    return h2
